# baseline (device time: 204964 ns/iter reference)
import jax
import jax.numpy as jnp
from jax import lax
from jax.experimental import pallas as pl
from jax.experimental.pallas import tpu as pltpu

N_DEV = 32
N_STAGES = 5

B, S, D = 2, 256, 512
DH = 64
H_LOC = 4
DQ = H_LOC * DH
EPS = 1e-5


def _layernorm(h):
    m = jnp.mean(h, axis=-1, keepdims=True)
    v = jnp.mean((h - m) ** 2, axis=-1, keepdims=True)
    return (h - m) * lax.rsqrt(v + EPS)


def kernel(x, Wq, Wk, Wv, Wo, t_emb, W_mod, W_ff1, W_ff2):
    def body(x_ref, wq_ref, wk_ref, wv_ref, wo_ref, temb_ref, wmod_ref,
             wff1_ref, wff2_ref, out_ref, attn_ref, acc_ref, recv_ref,
             send_sems, recv_sems):
        my_i = lax.axis_index("i")

        barrier_sem = pltpu.get_barrier_semaphore()
        for k in range(N_STAGES):
            pl.semaphore_signal(
                barrier_sem, inc=1,
                device_id=(my_i ^ (1 << k),),
                device_id_type=pl.DeviceIdType.MESH,
            )
        pl.semaphore_wait(barrier_sem, N_STAGES)

        def all_reduce(base):
            for k in range(N_STAGES):
                slot = base + k
                partner = my_i ^ (1 << k)
                rdma = pltpu.make_async_remote_copy(
                    src_ref=acc_ref,
                    dst_ref=recv_ref.at[slot],
                    send_sem=send_sems.at[slot],
                    recv_sem=recv_sems.at[slot],
                    device_id=(partner,),
                    device_id_type=pl.DeviceIdType.MESH,
                )
                rdma.start()
                rdma.wait_send()
                rdma.wait_recv()
                acc_ref[...] = acc_ref[...] + recv_ref[slot]

        mod = jnp.dot(temb_ref[...], wmod_ref[...],
                      preferred_element_type=jnp.float32)
        sa, sha, ga, sm, shm, gm = [mod[:, i * D:(i + 1) * D] for i in range(6)]

        x0 = x_ref[...]
        xa = _layernorm(x0) * (1.0 + sa[:, None, :]) + sha[:, None, :]
        xa2 = xa.reshape(B * S, D).astype(jnp.bfloat16)

        q = jnp.dot(xa2, wq_ref[...].astype(jnp.bfloat16),
                    preferred_element_type=jnp.float32)
        k_ = jnp.dot(xa2, wk_ref[...].astype(jnp.bfloat16),
                     preferred_element_type=jnp.float32)
        v_ = jnp.dot(xa2, wv_ref[...].astype(jnp.bfloat16),
                     preferred_element_type=jnp.float32)

        for b in range(B):
            for h in range(H_LOC):
                qh = q[b * S:(b + 1) * S, h * DH:(h + 1) * DH].astype(jnp.bfloat16)
                kh = k_[b * S:(b + 1) * S, h * DH:(h + 1) * DH].astype(jnp.bfloat16)
                vh = v_[b * S:(b + 1) * S, h * DH:(h + 1) * DH].astype(jnp.bfloat16)
                s = jnp.dot(qh, kh.T, preferred_element_type=jnp.float32) * 0.125
                s = s - jnp.max(s, axis=-1, keepdims=True)
                p = jnp.exp(s)
                p = p / jnp.sum(p, axis=-1, keepdims=True)
                attn_ref[b * S:(b + 1) * S, h * DH:(h + 1) * DH] = jnp.dot(
                    p.astype(jnp.bfloat16), vh, preferred_element_type=jnp.float32)

        acc_ref[...] = jnp.dot(attn_ref[...].astype(jnp.bfloat16),
                               wo_ref[...].astype(jnp.bfloat16),
                               preferred_element_type=jnp.float32)
        all_reduce(0)
        x1 = x0 + ga[:, None, :] * acc_ref[...].reshape(B, S, D)

        xm = _layernorm(x1) * (1.0 + sm[:, None, :]) + shm[:, None, :]
        xm2 = xm.reshape(B * S, D).astype(jnp.bfloat16)
        hmid = jnp.dot(xm2, wff1_ref[...].astype(jnp.bfloat16),
                       preferred_element_type=jnp.float32)
        hmid = hmid * (1.0 / (1.0 + jnp.exp(-hmid)))
        acc_ref[...] = jnp.dot(hmid.astype(jnp.bfloat16),
                               wff2_ref[...].astype(jnp.bfloat16),
                               preferred_element_type=jnp.float32)
        all_reduce(N_STAGES)
        out_ref[...] = x1 + gm[:, None, :] * acc_ref[...].reshape(B, S, D)

    vmem = pl.BlockSpec(memory_space=pltpu.VMEM)
    return pl.pallas_call(
        body,
        out_shape=jax.ShapeDtypeStruct((B, S, D), jnp.float32),
        in_specs=[vmem] * 9,
        out_specs=vmem,
        scratch_shapes=[
            pltpu.VMEM((B * S, DQ), jnp.float32),
            pltpu.VMEM((B * S, D), jnp.float32),
            pltpu.VMEM((2 * N_STAGES, B * S, D), jnp.float32),
            pltpu.SemaphoreType.DMA((2 * N_STAGES,)),
            pltpu.SemaphoreType.DMA((2 * N_STAGES,)),
        ],
        compiler_params=pltpu.CompilerParams(collective_id=0),
    )(x, Wq, Wk, Wv, Wo, t_emb, W_mod, W_ff1, W_ff2)


# device time: 120640 ns/iter; 1.6990x vs baseline; 1.6990x over previous
import jax
import jax.numpy as jnp
from jax import lax
from jax.experimental import pallas as pl
from jax.experimental.pallas import tpu as pltpu

N_DEV = 32
N_STAGES = 5

B, S, D = 2, 256, 512
DH = 64
H_LOC = 4
DQ = H_LOC * DH
EPS = 1e-5


def _layernorm(h):
    m = jnp.mean(h, axis=-1, keepdims=True)
    v = jnp.mean((h - m) ** 2, axis=-1, keepdims=True)
    return (h - m) * lax.rsqrt(v + EPS)


def kernel(x, Wq, Wk, Wv, Wo, t_emb, W_mod, W_ff1, W_ff2):
    def body(x_ref, wq_ref, wk_ref, wv_ref, wo_ref, temb_ref, wmod_ref,
             wff1_ref, wff2_ref, out_ref, attn_ref, acc_ref, recv_ref,
             send_sems, recv_sems):
        my_i = lax.axis_index("i")

        barrier_sem = pltpu.get_barrier_semaphore()
        for k in range(N_STAGES):
            pl.semaphore_signal(
                barrier_sem, inc=1,
                device_id=(my_i ^ (1 << k),),
                device_id_type=pl.DeviceIdType.MESH,
            )
        pl.semaphore_wait(barrier_sem, N_STAGES)

        def all_reduce(base):
            for k in range(N_STAGES):
                slot = base + k
                partner = my_i ^ (1 << k)
                rdma = pltpu.make_async_remote_copy(
                    src_ref=acc_ref,
                    dst_ref=recv_ref.at[slot],
                    send_sem=send_sems.at[slot],
                    recv_sem=recv_sems.at[slot],
                    device_id=(partner,),
                    device_id_type=pl.DeviceIdType.MESH,
                )
                rdma.start()
                rdma.wait_send()
                rdma.wait_recv()
                acc_ref[...] = (acc_ref[...].astype(jnp.float32)
                                + recv_ref[slot].astype(jnp.float32)
                                ).astype(jnp.bfloat16)

        mod = jnp.dot(temb_ref[...], wmod_ref[...],
                      preferred_element_type=jnp.float32)
        sa, sha, ga, sm, shm, gm = [mod[:, i * D:(i + 1) * D] for i in range(6)]

        x0 = x_ref[...]
        xa = _layernorm(x0) * (1.0 + sa[:, None, :]) + sha[:, None, :]
        xa2 = xa.reshape(B * S, D).astype(jnp.bfloat16)

        q = jnp.dot(xa2, wq_ref[...].astype(jnp.bfloat16),
                    preferred_element_type=jnp.float32)
        k_ = jnp.dot(xa2, wk_ref[...].astype(jnp.bfloat16),
                     preferred_element_type=jnp.float32)
        v_ = jnp.dot(xa2, wv_ref[...].astype(jnp.bfloat16),
                     preferred_element_type=jnp.float32)

        for b in range(B):
            for h in range(H_LOC):
                qh = q[b * S:(b + 1) * S, h * DH:(h + 1) * DH].astype(jnp.bfloat16)
                kh = k_[b * S:(b + 1) * S, h * DH:(h + 1) * DH].astype(jnp.bfloat16)
                vh = v_[b * S:(b + 1) * S, h * DH:(h + 1) * DH].astype(jnp.bfloat16)
                s = jnp.dot(qh, kh.T, preferred_element_type=jnp.float32) * 0.125
                s = s - jnp.max(s, axis=-1, keepdims=True)
                p = jnp.exp(s)
                p = p / jnp.sum(p, axis=-1, keepdims=True)
                attn_ref[b * S:(b + 1) * S, h * DH:(h + 1) * DH] = jnp.dot(
                    p.astype(jnp.bfloat16), vh, preferred_element_type=jnp.float32)

        acc_ref[...] = jnp.dot(attn_ref[...].astype(jnp.bfloat16),
                               wo_ref[...].astype(jnp.bfloat16),
                               preferred_element_type=jnp.float32
                               ).astype(jnp.bfloat16)
        all_reduce(0)
        x1 = x0 + ga[:, None, :] * acc_ref[...].astype(jnp.float32).reshape(B, S, D)

        xm = _layernorm(x1) * (1.0 + sm[:, None, :]) + shm[:, None, :]
        xm2 = xm.reshape(B * S, D).astype(jnp.bfloat16)
        hmid = jnp.dot(xm2, wff1_ref[...].astype(jnp.bfloat16),
                       preferred_element_type=jnp.float32)
        hmid = hmid * (1.0 / (1.0 + jnp.exp(-hmid)))
        acc_ref[...] = jnp.dot(hmid.astype(jnp.bfloat16),
                               wff2_ref[...].astype(jnp.bfloat16),
                               preferred_element_type=jnp.float32
                               ).astype(jnp.bfloat16)
        all_reduce(N_STAGES)
        out_ref[...] = x1 + gm[:, None, :] * acc_ref[...].astype(jnp.float32).reshape(B, S, D)

    vmem = pl.BlockSpec(memory_space=pltpu.VMEM)
    return pl.pallas_call(
        body,
        out_shape=jax.ShapeDtypeStruct((B, S, D), jnp.float32),
        in_specs=[vmem] * 9,
        out_specs=vmem,
        scratch_shapes=[
            pltpu.VMEM((B * S, DQ), jnp.float32),
            pltpu.VMEM((B * S, D), jnp.bfloat16),
            pltpu.VMEM((2 * N_STAGES, B * S, D), jnp.bfloat16),
            pltpu.SemaphoreType.DMA((2 * N_STAGES,)),
            pltpu.SemaphoreType.DMA((2 * N_STAGES,)),
        ],
        compiler_params=pltpu.CompilerParams(collective_id=0),
    )(x, Wq, Wk, Wv, Wo, t_emb, W_mod, W_ff1, W_ff2)


# device time: 96591 ns/iter; 2.1220x vs baseline; 1.2490x over previous
import jax
import jax.numpy as jnp
from jax import lax
from jax.experimental import pallas as pl
from jax.experimental.pallas import tpu as pltpu

N_DEV = 32
N_STAGES = 5

B, S, D = 2, 256, 512
DH = 64
H_LOC = 4
DQ = H_LOC * DH
EPS = 1e-5

C = 4
R = (B * S) // C
N_SLOTS = 2 * N_STAGES * C


def _layernorm(h):
    m = jnp.mean(h, axis=-1, keepdims=True)
    v = jnp.mean((h - m) ** 2, axis=-1, keepdims=True)
    return (h - m) * lax.rsqrt(v + EPS)


def kernel(x, Wq, Wk, Wv, Wo, t_emb, W_mod, W_ff1, W_ff2):
    def body(x_ref, wq_ref, wk_ref, wv_ref, wo_ref, temb_ref, wmod_ref,
             wff1_ref, wff2_ref, out_ref, attn_ref, acc_ref, recv_ref,
             send_sems, recv_sems):
        my_i = lax.axis_index("i")

        barrier_sem = pltpu.get_barrier_semaphore()
        for k in range(N_STAGES):
            pl.semaphore_signal(
                barrier_sem, inc=1,
                device_id=(my_i ^ (1 << k),),
                device_id_type=pl.DeviceIdType.MESH,
            )
        pl.semaphore_wait(barrier_sem, N_STAGES)

        def all_reduce(base):
            def slot(k, c):
                return (base * N_STAGES + k) * C + c

            def make(k, c):
                return pltpu.make_async_remote_copy(
                    src_ref=acc_ref.at[pl.ds(c * R, R), :],
                    dst_ref=recv_ref.at[slot(k, c)],
                    send_sem=send_sems.at[slot(k, c)],
                    recv_sem=recv_sems.at[slot(k, c)],
                    device_id=(my_i ^ (1 << k),),
                    device_id_type=pl.DeviceIdType.MESH,
                )

            for c in range(C):
                make(0, c).start()
            for k in range(N_STAGES):
                for c in range(C):
                    rdma = make(k, c)
                    rdma.wait_send()
                    rdma.wait_recv()
                    acc_ref[c * R:(c + 1) * R, :] = (
                        acc_ref[c * R:(c + 1) * R, :].astype(jnp.float32)
                        + recv_ref[slot(k, c)].astype(jnp.float32)
                    ).astype(jnp.bfloat16)
                    if k + 1 < N_STAGES:
                        make(k + 1, c).start()

        mod = jnp.dot(temb_ref[...], wmod_ref[...],
                      preferred_element_type=jnp.float32)
        sa, sha, ga, sm, shm, gm = [mod[:, i * D:(i + 1) * D] for i in range(6)]

        x0 = x_ref[...]
        xa = _layernorm(x0) * (1.0 + sa[:, None, :]) + sha[:, None, :]
        xa2 = xa.reshape(B * S, D).astype(jnp.bfloat16)

        q = jnp.dot(xa2, wq_ref[...].astype(jnp.bfloat16),
                    preferred_element_type=jnp.float32)
        k_ = jnp.dot(xa2, wk_ref[...].astype(jnp.bfloat16),
                     preferred_element_type=jnp.float32)
        v_ = jnp.dot(xa2, wv_ref[...].astype(jnp.bfloat16),
                     preferred_element_type=jnp.float32)

        for b in range(B):
            for h in range(H_LOC):
                qh = q[b * S:(b + 1) * S, h * DH:(h + 1) * DH].astype(jnp.bfloat16)
                kh = k_[b * S:(b + 1) * S, h * DH:(h + 1) * DH].astype(jnp.bfloat16)
                vh = v_[b * S:(b + 1) * S, h * DH:(h + 1) * DH].astype(jnp.bfloat16)
                s = jnp.dot(qh, kh.T, preferred_element_type=jnp.float32) * 0.125
                s = s - jnp.max(s, axis=-1, keepdims=True)
                p = jnp.exp(s)
                p = p / jnp.sum(p, axis=-1, keepdims=True)
                attn_ref[b * S:(b + 1) * S, h * DH:(h + 1) * DH] = jnp.dot(
                    p.astype(jnp.bfloat16), vh, preferred_element_type=jnp.float32)

        acc_ref[...] = jnp.dot(attn_ref[...].astype(jnp.bfloat16),
                               wo_ref[...].astype(jnp.bfloat16),
                               preferred_element_type=jnp.float32
                               ).astype(jnp.bfloat16)
        all_reduce(0)
        x1 = x0 + ga[:, None, :] * acc_ref[...].astype(jnp.float32).reshape(B, S, D)

        xm = _layernorm(x1) * (1.0 + sm[:, None, :]) + shm[:, None, :]
        xm2 = xm.reshape(B * S, D).astype(jnp.bfloat16)
        hmid = jnp.dot(xm2, wff1_ref[...].astype(jnp.bfloat16),
                       preferred_element_type=jnp.float32)
        hmid = hmid * (1.0 / (1.0 + jnp.exp(-hmid)))
        acc_ref[...] = jnp.dot(hmid.astype(jnp.bfloat16),
                               wff2_ref[...].astype(jnp.bfloat16),
                               preferred_element_type=jnp.float32
                               ).astype(jnp.bfloat16)
        all_reduce(1)
        out_ref[...] = x1 + gm[:, None, :] * acc_ref[...].astype(jnp.float32).reshape(B, S, D)

    vmem = pl.BlockSpec(memory_space=pltpu.VMEM)
    return pl.pallas_call(
        body,
        out_shape=jax.ShapeDtypeStruct((B, S, D), jnp.float32),
        in_specs=[vmem] * 9,
        out_specs=vmem,
        scratch_shapes=[
            pltpu.VMEM((B * S, DQ), jnp.float32),
            pltpu.VMEM((B * S, D), jnp.bfloat16),
            pltpu.VMEM((N_SLOTS, R, D), jnp.bfloat16),
            pltpu.SemaphoreType.DMA((N_SLOTS,)),
            pltpu.SemaphoreType.DMA((N_SLOTS,)),
        ],
        compiler_params=pltpu.CompilerParams(collective_id=0),
    )(x, Wq, Wk, Wv, Wo, t_emb, W_mod, W_ff1, W_ff2)


# device time: 56805 ns/iter; 3.6082x vs baseline; 1.7004x over previous
import jax
import jax.numpy as jnp
from jax import lax
from jax.experimental import pallas as pl
from jax.experimental.pallas import tpu as pltpu

N_DEV = 32
N_STAGES = 5

B, S, D = 2, 256, 512
DH = 64
H_LOC = 4
DQ = H_LOC * DH
EPS = 1e-5

CH = (B * S) // N_DEV


def _layernorm(h):
    m = jnp.mean(h, axis=-1, keepdims=True)
    v = jnp.mean((h - m) ** 2, axis=-1, keepdims=True)
    return (h - m) * lax.rsqrt(v + EPS)


def kernel(x, Wq, Wk, Wv, Wo, t_emb, W_mod, W_ff1, W_ff2):
    def body(x_ref, wq_ref, wk_ref, wv_ref, wo_ref, temb_ref, wmod_ref,
             wff1_ref, wff2_ref, out_ref, attn_ref, acc_ref, scat_ref,
             send_sems, recv_sems, ag_send_sems, ag_recv_sems):
        my_i = lax.axis_index("i")

        barrier_sem = pltpu.get_barrier_semaphore()
        for r in range(1, N_DEV):
            pl.semaphore_signal(
                barrier_sem, inc=1,
                device_id=((my_i + r) % N_DEV,),
                device_id_type=pl.DeviceIdType.MESH,
            )
        pl.semaphore_wait(barrier_sem, N_DEV - 1)

        def scatter_rdma(r):
            j = (my_i + r) % N_DEV
            return pltpu.make_async_remote_copy(
                src_ref=acc_ref.at[pl.ds(j * CH, CH), :],
                dst_ref=scat_ref.at[r],
                send_sem=send_sems.at[r],
                recv_sem=recv_sems.at[r],
                device_id=(j,),
                device_id_type=pl.DeviceIdType.MESH,
            )

        def gather_rdma(r):
            j = (my_i + r) % N_DEV
            return pltpu.make_async_remote_copy(
                src_ref=acc_ref.at[pl.ds(my_i * CH, CH), :],
                dst_ref=acc_ref.at[pl.ds(my_i * CH, CH), :],
                send_sem=ag_send_sems.at[r],
                recv_sem=ag_recv_sems.at[r],
                device_id=(j,),
                device_id_type=pl.DeviceIdType.MESH,
            )

        def all_reduce():
            scat_ref[0] = acc_ref[pl.ds(my_i * CH, CH), :]
            for r in range(1, N_DEV):
                scatter_rdma(r).start()
            for r in range(1, N_DEV):
                scatter_rdma(r).wait_recv()
            reduced = jnp.sum(scat_ref[...].astype(jnp.float32), axis=0)
            acc_ref[pl.ds(my_i * CH, CH), :] = reduced.astype(jnp.bfloat16)
            for r in range(1, N_DEV):
                gather_rdma(r).start()
            for r in range(1, N_DEV):
                gather_rdma(r).wait_recv()
            for r in range(1, N_DEV):
                scatter_rdma(r).wait_send()
                gather_rdma(r).wait_send()

        mod = jnp.dot(temb_ref[...], wmod_ref[...],
                      preferred_element_type=jnp.float32)
        sa, sha, ga, sm, shm, gm = [mod[:, i * D:(i + 1) * D] for i in range(6)]

        x0 = x_ref[...]
        xa = _layernorm(x0) * (1.0 + sa[:, None, :]) + sha[:, None, :]
        xa2 = xa.reshape(B * S, D).astype(jnp.bfloat16)

        q = jnp.dot(xa2, wq_ref[...].astype(jnp.bfloat16),
                    preferred_element_type=jnp.float32)
        k_ = jnp.dot(xa2, wk_ref[...].astype(jnp.bfloat16),
                     preferred_element_type=jnp.float32)
        v_ = jnp.dot(xa2, wv_ref[...].astype(jnp.bfloat16),
                     preferred_element_type=jnp.float32)

        for b in range(B):
            for h in range(H_LOC):
                qh = q[b * S:(b + 1) * S, h * DH:(h + 1) * DH].astype(jnp.bfloat16)
                kh = k_[b * S:(b + 1) * S, h * DH:(h + 1) * DH].astype(jnp.bfloat16)
                vh = v_[b * S:(b + 1) * S, h * DH:(h + 1) * DH].astype(jnp.bfloat16)
                s = jnp.dot(qh, kh.T, preferred_element_type=jnp.float32) * 0.125
                s = s - jnp.max(s, axis=-1, keepdims=True)
                p = jnp.exp(s)
                p = p / jnp.sum(p, axis=-1, keepdims=True)
                attn_ref[b * S:(b + 1) * S, h * DH:(h + 1) * DH] = jnp.dot(
                    p.astype(jnp.bfloat16), vh, preferred_element_type=jnp.float32)

        acc_ref[...] = jnp.dot(attn_ref[...].astype(jnp.bfloat16),
                               wo_ref[...].astype(jnp.bfloat16),
                               preferred_element_type=jnp.float32
                               ).astype(jnp.bfloat16)
        all_reduce()
        x1 = x0 + ga[:, None, :] * acc_ref[...].astype(jnp.float32).reshape(B, S, D)

        xm = _layernorm(x1) * (1.0 + sm[:, None, :]) + shm[:, None, :]
        xm2 = xm.reshape(B * S, D).astype(jnp.bfloat16)
        hmid = jnp.dot(xm2, wff1_ref[...].astype(jnp.bfloat16),
                       preferred_element_type=jnp.float32)
        hmid = hmid * (1.0 / (1.0 + jnp.exp(-hmid)))
        acc_ref[...] = jnp.dot(hmid.astype(jnp.bfloat16),
                               wff2_ref[...].astype(jnp.bfloat16),
                               preferred_element_type=jnp.float32
                               ).astype(jnp.bfloat16)
        all_reduce()
        out_ref[...] = x1 + gm[:, None, :] * acc_ref[...].astype(jnp.float32).reshape(B, S, D)

    vmem = pl.BlockSpec(memory_space=pltpu.VMEM)
    return pl.pallas_call(
        body,
        out_shape=jax.ShapeDtypeStruct((B, S, D), jnp.float32),
        in_specs=[vmem] * 9,
        out_specs=vmem,
        scratch_shapes=[
            pltpu.VMEM((B * S, DQ), jnp.float32),
            pltpu.VMEM((B * S, D), jnp.bfloat16),
            pltpu.VMEM((N_DEV, CH, D), jnp.bfloat16),
            pltpu.SemaphoreType.DMA((N_DEV,)),
            pltpu.SemaphoreType.DMA((N_DEV,)),
            pltpu.SemaphoreType.DMA((N_DEV,)),
            pltpu.SemaphoreType.DMA((N_DEV,)),
        ],
        compiler_params=pltpu.CompilerParams(collective_id=0),
    )(x, Wq, Wk, Wv, Wo, t_emb, W_mod, W_ff1, W_ff2)


# device time: 52295 ns/iter; 3.9194x vs baseline; 1.0862x over previous
import jax
import jax.numpy as jnp
from jax import lax
from jax.experimental import pallas as pl
from jax.experimental.pallas import tpu as pltpu

N_DEV = 32

B, S, D = 2, 256, 512
DH = 64
H_LOC = 4
DQ = H_LOC * DH
EPS = 1e-5

CH = (B * S) // N_DEV
HALF = B * S // 2


def _layernorm(h):
    m = jnp.mean(h, axis=-1, keepdims=True)
    v = jnp.mean((h - m) ** 2, axis=-1, keepdims=True)
    return (h - m) * lax.rsqrt(v + EPS)


def kernel(x, Wq, Wk, Wv, Wo, t_emb, W_mod, W_ff1, W_ff2):
    def body(x_ref, wq_ref, wk_ref, wv_ref, wo_ref, temb_ref, wmod_ref,
             wff1_ref, wff2_ref, out_ref, attn_ref, acc_ref, x1_ref, scat_ref,
             send_sems, recv_sems, ag_send_sems, ag_recv_sems):
        my_i = lax.axis_index("i")

        barrier_sem = pltpu.get_barrier_semaphore()
        for r in range(1, N_DEV):
            pl.semaphore_signal(
                barrier_sem, inc=1,
                device_id=((my_i + r) % N_DEV,),
                device_id_type=pl.DeviceIdType.MESH,
            )

        def scatter_to(j):
            r = (j - my_i) % N_DEV
            return pltpu.make_async_remote_copy(
                src_ref=acc_ref.at[pl.ds(j * CH, CH), :],
                dst_ref=scat_ref.at[r],
                send_sem=send_sems.at[r],
                recv_sem=recv_sems.at[r],
                device_id=(j,),
                device_id_type=pl.DeviceIdType.MESH,
            )

        def start_scatter(j):
            @pl.when(j != my_i)
            def _():
                scatter_to(j).start()

        def scatter_rel(r):
            j = (my_i + r) % N_DEV
            return pltpu.make_async_remote_copy(
                src_ref=acc_ref.at[pl.ds(j * CH, CH), :],
                dst_ref=scat_ref.at[r],
                send_sem=send_sems.at[r],
                recv_sem=recv_sems.at[r],
                device_id=(j,),
                device_id_type=pl.DeviceIdType.MESH,
            )

        def gather_rdma(r):
            j = (my_i + r) % N_DEV
            return pltpu.make_async_remote_copy(
                src_ref=acc_ref.at[pl.ds(my_i * CH, CH), :],
                dst_ref=acc_ref.at[pl.ds(my_i * CH, CH), :],
                send_sem=ag_send_sems.at[r],
                recv_sem=ag_recv_sems.at[r],
                device_id=(j,),
                device_id_type=pl.DeviceIdType.MESH,
            )

        def wait_ag_from(j):
            @pl.when(j != my_i)
            def _():
                r = (my_i - j) % N_DEV
                pltpu.make_async_remote_copy(
                    src_ref=acc_ref.at[pl.ds(j * CH, CH), :],
                    dst_ref=acc_ref.at[pl.ds(j * CH, CH), :],
                    send_sem=ag_send_sems.at[r],
                    recv_sem=ag_recv_sems.at[r],
                    device_id=(j,),
                    device_id_type=pl.DeviceIdType.MESH,
                ).wait_recv()

        mod = jnp.dot(temb_ref[...], wmod_ref[...],
                      preferred_element_type=jnp.float32)
        sa, sha, ga, sm, shm, gm = [mod[:, i * D:(i + 1) * D] for i in range(6)]

        x0 = x_ref[...]
        xa = _layernorm(x0) * (1.0 + sa[:, None, :]) + sha[:, None, :]
        xa2 = xa.reshape(B * S, D).astype(jnp.bfloat16)

        q = jnp.dot(xa2, wq_ref[...].astype(jnp.bfloat16),
                    preferred_element_type=jnp.float32)
        k_ = jnp.dot(xa2, wk_ref[...].astype(jnp.bfloat16),
                     preferred_element_type=jnp.float32)
        v_ = jnp.dot(xa2, wv_ref[...].astype(jnp.bfloat16),
                     preferred_element_type=jnp.float32)
        wo_bf = wo_ref[...].astype(jnp.bfloat16)

        for b in range(B):
            for h in range(H_LOC):
                qh = q[b * S:(b + 1) * S, h * DH:(h + 1) * DH].astype(jnp.bfloat16)
                kh = k_[b * S:(b + 1) * S, h * DH:(h + 1) * DH].astype(jnp.bfloat16)
                vh = v_[b * S:(b + 1) * S, h * DH:(h + 1) * DH].astype(jnp.bfloat16)
                s = jnp.dot(qh, kh.T, preferred_element_type=jnp.float32) * 0.125
                s = s - jnp.max(s, axis=-1, keepdims=True)
                p = jnp.exp(s)
                p = p / jnp.sum(p, axis=-1, keepdims=True)
                attn_ref[b * S:(b + 1) * S, h * DH:(h + 1) * DH] = jnp.dot(
                    p.astype(jnp.bfloat16), vh, preferred_element_type=jnp.float32)
            acc_ref[b * S:(b + 1) * S, :] = jnp.dot(
                attn_ref[b * S:(b + 1) * S, :].astype(jnp.bfloat16), wo_bf,
                preferred_element_type=jnp.float32).astype(jnp.bfloat16)
            if b == 0:
                pl.semaphore_wait(barrier_sem, N_DEV - 1)
            for j in range(b * (N_DEV // B), (b + 1) * (N_DEV // B)):
                start_scatter(j)

        scat_ref[0] = acc_ref[pl.ds(my_i * CH, CH), :]
        for r in range(1, N_DEV):
            scatter_rel(r).wait_recv()
        reduced = jnp.sum(scat_ref[...].astype(jnp.float32), axis=0)
        acc_ref[pl.ds(my_i * CH, CH), :] = reduced.astype(jnp.bfloat16)
        for r in range(1, N_DEV):
            gather_rdma(r).start()

        x0f = x0.reshape(B * S, D)
        for half in range(B):
            rows = slice(half * HALF, (half + 1) * HALF)
            for j in range(half * (N_DEV // B), (half + 1) * (N_DEV // B)):
                wait_ag_from(j)
            x1h = x0f[rows, :] + ga[half:half + 1, :] * acc_ref[rows, :].astype(jnp.float32)
            x1_ref[rows, :] = x1h
            xmh = (_layernorm(x1h) * (1.0 + sm[half:half + 1, :])
                   + shm[half:half + 1, :]).astype(jnp.bfloat16)
            hh = jnp.dot(xmh, wff1_ref[...].astype(jnp.bfloat16),
                         preferred_element_type=jnp.float32)
            hh = hh * (1.0 / (1.0 + jnp.exp(-hh)))
            p2h = jnp.dot(hh.astype(jnp.bfloat16),
                          wff2_ref[...].astype(jnp.bfloat16),
                          preferred_element_type=jnp.float32).astype(jnp.bfloat16)
            if half == 0:
                for r in range(1, N_DEV):
                    scatter_rel(r).wait_send()
                    gather_rdma(r).wait_send()
            acc_ref[rows, :] = p2h
            for j in range(half * (N_DEV // B), (half + 1) * (N_DEV // B)):
                start_scatter(j)

        scat_ref[0] = acc_ref[pl.ds(my_i * CH, CH), :]
        for r in range(1, N_DEV):
            scatter_rel(r).wait_recv()
        reduced2 = jnp.sum(scat_ref[...].astype(jnp.float32), axis=0)
        acc_ref[pl.ds(my_i * CH, CH), :] = reduced2.astype(jnp.bfloat16)
        for r in range(1, N_DEV):
            gather_rdma(r).start()
        for r in range(1, N_DEV):
            gather_rdma(r).wait_recv()

        out_ref[...] = (x1_ref[...].reshape(B, S, D)
                        + gm[:, None, :] * acc_ref[...].astype(jnp.float32).reshape(B, S, D))

        for r in range(1, N_DEV):
            scatter_rel(r).wait_send()
            gather_rdma(r).wait_send()

    vmem = pl.BlockSpec(memory_space=pltpu.VMEM)
    return pl.pallas_call(
        body,
        out_shape=jax.ShapeDtypeStruct((B, S, D), jnp.float32),
        in_specs=[vmem] * 9,
        out_specs=vmem,
        scratch_shapes=[
            pltpu.VMEM((B * S, DQ), jnp.float32),
            pltpu.VMEM((B * S, D), jnp.bfloat16),
            pltpu.VMEM((B * S, D), jnp.float32),
            pltpu.VMEM((N_DEV, CH, D), jnp.bfloat16),
            pltpu.SemaphoreType.DMA((N_DEV,)),
            pltpu.SemaphoreType.DMA((N_DEV,)),
            pltpu.SemaphoreType.DMA((N_DEV,)),
            pltpu.SemaphoreType.DMA((N_DEV,)),
        ],
        compiler_params=pltpu.CompilerParams(collective_id=0),
    )(x, Wq, Wk, Wv, Wo, t_emb, W_mod, W_ff1, W_ff2)


# device time: 51839 ns/iter; 3.9539x vs baseline; 1.0088x over previous
import jax
import jax.numpy as jnp
from jax import lax
from jax.experimental import pallas as pl
from jax.experimental.pallas import tpu as pltpu

N_DEV = 32

B, S, D = 2, 256, 512
DH = 64
H_LOC = 4
DQ = H_LOC * DH
EPS = 1e-5

CH = (B * S) // N_DEV
HALF = B * S // 2


def _layernorm(h):
    m = jnp.mean(h, axis=-1, keepdims=True)
    v = jnp.mean((h - m) ** 2, axis=-1, keepdims=True)
    return (h - m) * lax.rsqrt(v + EPS)


def kernel(x, Wq, Wk, Wv, Wo, t_emb, W_mod, W_ff1, W_ff2):
    def body(x_ref, wq_ref, wk_ref, wv_ref, wo_ref, temb_ref, wmod_ref,
             wff1_ref, wff2_ref, out_ref, attn_ref, acc_ref, x1_ref, scat_ref,
             send_sems, recv_sems, ag_send_sems, ag_recv_sems):
        my_i = lax.axis_index("i")

        barrier_sem = pltpu.get_barrier_semaphore()
        for r in range(1, N_DEV):
            pl.semaphore_signal(
                barrier_sem, inc=1,
                device_id=((my_i + r) % N_DEV,),
                device_id_type=pl.DeviceIdType.MESH,
            )

        def scatter_to(j):
            r = (j - my_i) % N_DEV
            return pltpu.make_async_remote_copy(
                src_ref=acc_ref.at[pl.ds(j * CH, CH), :],
                dst_ref=scat_ref.at[r],
                send_sem=send_sems.at[r],
                recv_sem=recv_sems.at[r],
                device_id=(j,),
                device_id_type=pl.DeviceIdType.MESH,
            )

        def start_scatter(j):
            @pl.when(j != my_i)
            def _():
                scatter_to(j).start()

        def scatter_rel(r):
            j = (my_i + r) % N_DEV
            return pltpu.make_async_remote_copy(
                src_ref=acc_ref.at[pl.ds(j * CH, CH), :],
                dst_ref=scat_ref.at[r],
                send_sem=send_sems.at[r],
                recv_sem=recv_sems.at[r],
                device_id=(j,),
                device_id_type=pl.DeviceIdType.MESH,
            )

        def gather_rdma(r):
            j = (my_i + r) % N_DEV
            return pltpu.make_async_remote_copy(
                src_ref=acc_ref.at[pl.ds(my_i * CH, CH), :],
                dst_ref=acc_ref.at[pl.ds(my_i * CH, CH), :],
                send_sem=ag_send_sems.at[r],
                recv_sem=ag_recv_sems.at[r],
                device_id=(j,),
                device_id_type=pl.DeviceIdType.MESH,
            )

        def wait_ag_from(j):
            @pl.when(j != my_i)
            def _():
                r = (my_i - j) % N_DEV
                pltpu.make_async_remote_copy(
                    src_ref=acc_ref.at[pl.ds(j * CH, CH), :],
                    dst_ref=acc_ref.at[pl.ds(j * CH, CH), :],
                    send_sem=ag_send_sems.at[r],
                    recv_sem=ag_recv_sems.at[r],
                    device_id=(j,),
                    device_id_type=pl.DeviceIdType.MESH,
                ).wait_recv()

        mod = jnp.dot(temb_ref[...], wmod_ref[...],
                      preferred_element_type=jnp.float32)
        sa, sha, ga, sm, shm, gm = [mod[:, i * D:(i + 1) * D] for i in range(6)]

        x0 = x_ref[...]
        xa = _layernorm(x0) * (1.0 + sa[:, None, :]) + sha[:, None, :]
        xa2 = xa.reshape(B * S, D).astype(jnp.bfloat16)

        q = jnp.dot(xa2, wq_ref[...].astype(jnp.bfloat16),
                    preferred_element_type=jnp.float32)
        k_ = jnp.dot(xa2, wk_ref[...].astype(jnp.bfloat16),
                     preferred_element_type=jnp.float32)
        v_ = jnp.dot(xa2, wv_ref[...].astype(jnp.bfloat16),
                     preferred_element_type=jnp.float32)
        wo_bf = wo_ref[...].astype(jnp.bfloat16)

        for b in range(B):
            for h in range(H_LOC):
                qh = q[b * S:(b + 1) * S, h * DH:(h + 1) * DH].astype(jnp.bfloat16)
                kh = k_[b * S:(b + 1) * S, h * DH:(h + 1) * DH].astype(jnp.bfloat16)
                vh = v_[b * S:(b + 1) * S, h * DH:(h + 1) * DH].astype(jnp.bfloat16)
                s = jnp.dot(qh, kh.T, preferred_element_type=jnp.float32) * 0.125
                s = s - jnp.max(s, axis=-1, keepdims=True)
                p = jnp.exp(s)
                p = p / jnp.sum(p, axis=-1, keepdims=True)
                attn_ref[b * S:(b + 1) * S, h * DH:(h + 1) * DH] = jnp.dot(
                    p.astype(jnp.bfloat16), vh, preferred_element_type=jnp.float32)
            acc_ref[b * S:(b + 1) * S, :] = jnp.dot(
                attn_ref[b * S:(b + 1) * S, :].astype(jnp.bfloat16), wo_bf,
                preferred_element_type=jnp.float32).astype(jnp.bfloat16)
            if b == 0:
                pl.semaphore_wait(barrier_sem, N_DEV - 1)
            for j in range(b * (N_DEV // B), (b + 1) * (N_DEV // B)):
                start_scatter(j)

        scat_ref[0] = acc_ref[pl.ds(my_i * CH, CH), :]
        for r in range(1, N_DEV):
            scatter_rel(r).wait_recv()
        reduced = jnp.sum(scat_ref[...].astype(jnp.float32), axis=0)
        acc_ref[pl.ds(my_i * CH, CH), :] = reduced.astype(jnp.bfloat16)
        for r in range(1, N_DEV):
            gather_rdma(r).start()

        x0f = x0.reshape(B * S, D)
        n_grp = 4
        own_per = N_DEV // n_grp
        rows_per = B * S // n_grp
        wff1_bf = wff1_ref[...].astype(jnp.bfloat16)
        wff2_bf = wff2_ref[...].astype(jnp.bfloat16)
        for g in range(n_grp):
            bat = g // (n_grp // B)
            rows = slice(g * rows_per, (g + 1) * rows_per)
            for j in range(g * own_per, (g + 1) * own_per):
                wait_ag_from(j)
            x1h = x0f[rows, :] + ga[bat:bat + 1, :] * acc_ref[rows, :].astype(jnp.float32)
            x1_ref[rows, :] = x1h
            xmh = (_layernorm(x1h) * (1.0 + sm[bat:bat + 1, :])
                   + shm[bat:bat + 1, :]).astype(jnp.bfloat16)
            hh = jnp.dot(xmh, wff1_bf,
                         preferred_element_type=jnp.float32)
            hh = hh * (1.0 / (1.0 + jnp.exp(-hh)))
            p2h = jnp.dot(hh.astype(jnp.bfloat16), wff2_bf,
                          preferred_element_type=jnp.float32).astype(jnp.bfloat16)
            if g == 0:
                for r in range(1, N_DEV):
                    scatter_rel(r).wait_send()
                    gather_rdma(r).wait_send()
            acc_ref[rows, :] = p2h
            for j in range(g * own_per, (g + 1) * own_per):
                start_scatter(j)

        scat_ref[0] = acc_ref[pl.ds(my_i * CH, CH), :]
        for r in range(1, N_DEV):
            scatter_rel(r).wait_recv()
        reduced2 = jnp.sum(scat_ref[...].astype(jnp.float32), axis=0)
        acc_ref[pl.ds(my_i * CH, CH), :] = reduced2.astype(jnp.bfloat16)
        for r in range(1, N_DEV):
            gather_rdma(r).start()
        for r in range(1, N_DEV):
            gather_rdma(r).wait_recv()

        out_ref[...] = (x1_ref[...].reshape(B, S, D)
                        + gm[:, None, :] * acc_ref[...].astype(jnp.float32).reshape(B, S, D))

        for r in range(1, N_DEV):
            scatter_rel(r).wait_send()
            gather_rdma(r).wait_send()

    vmem = pl.BlockSpec(memory_space=pltpu.VMEM)
    return pl.pallas_call(
        body,
        out_shape=jax.ShapeDtypeStruct((B, S, D), jnp.float32),
        in_specs=[vmem] * 9,
        out_specs=vmem,
        scratch_shapes=[
            pltpu.VMEM((B * S, DQ), jnp.float32),
            pltpu.VMEM((B * S, D), jnp.bfloat16),
            pltpu.VMEM((B * S, D), jnp.float32),
            pltpu.VMEM((N_DEV, CH, D), jnp.bfloat16),
            pltpu.SemaphoreType.DMA((N_DEV,)),
            pltpu.SemaphoreType.DMA((N_DEV,)),
            pltpu.SemaphoreType.DMA((N_DEV,)),
            pltpu.SemaphoreType.DMA((N_DEV,)),
        ],
        compiler_params=pltpu.CompilerParams(collective_id=0),
    )(x, Wq, Wk, Wv, Wo, t_emb, W_mod, W_ff1, W_ff2)


# device time: 51682 ns/iter; 3.9659x vs baseline; 1.0030x over previous
import jax
import jax.numpy as jnp
from jax import lax
from jax.experimental import pallas as pl
from jax.experimental.pallas import tpu as pltpu

N_DEV = 32

B, S, D = 2, 256, 512
DH = 64
H_LOC = 4
DQ = H_LOC * DH
EPS = 1e-5

CH = (B * S) // N_DEV
HALF = B * S // 2


def _layernorm(h):
    m = jnp.mean(h, axis=-1, keepdims=True)
    v = jnp.mean((h - m) ** 2, axis=-1, keepdims=True)
    return (h - m) * lax.rsqrt(v + EPS)


def kernel(x, Wq, Wk, Wv, Wo, t_emb, W_mod, W_ff1, W_ff2):
    def body(x_ref, wq_ref, wk_ref, wv_ref, wo_ref, temb_ref, wmod_ref,
             wff1_ref, wff2_ref, out_ref, attn_ref, acc_ref, x1_ref, scat_ref,
             send_sems, recv_sems, ag_send_sems, ag_recv_sems):
        my_i = lax.axis_index("i")

        barrier_sem = pltpu.get_barrier_semaphore()
        for r in range(1, N_DEV):
            pl.semaphore_signal(
                barrier_sem, inc=1,
                device_id=((my_i + r) % N_DEV,),
                device_id_type=pl.DeviceIdType.MESH,
            )

        def scatter_to(j):
            r = (j - my_i) % N_DEV
            return pltpu.make_async_remote_copy(
                src_ref=acc_ref.at[pl.ds(j * CH, CH), :],
                dst_ref=scat_ref.at[r],
                send_sem=send_sems.at[r],
                recv_sem=recv_sems.at[r],
                device_id=(j,),
                device_id_type=pl.DeviceIdType.MESH,
            )

        def start_scatter(j):
            @pl.when(j != my_i)
            def _():
                scatter_to(j).start()

        def scatter_rel(r):
            j = (my_i + r) % N_DEV
            return pltpu.make_async_remote_copy(
                src_ref=acc_ref.at[pl.ds(j * CH, CH), :],
                dst_ref=scat_ref.at[r],
                send_sem=send_sems.at[r],
                recv_sem=recv_sems.at[r],
                device_id=(j,),
                device_id_type=pl.DeviceIdType.MESH,
            )

        def gather_rdma(r):
            j = (my_i + r) % N_DEV
            return pltpu.make_async_remote_copy(
                src_ref=acc_ref.at[pl.ds(my_i * CH, CH), :],
                dst_ref=acc_ref.at[pl.ds(my_i * CH, CH), :],
                send_sem=ag_send_sems.at[r],
                recv_sem=ag_recv_sems.at[r],
                device_id=(j,),
                device_id_type=pl.DeviceIdType.MESH,
            )

        def wait_ag_from(j):
            @pl.when(j != my_i)
            def _():
                r = (my_i - j) % N_DEV
                pltpu.make_async_remote_copy(
                    src_ref=acc_ref.at[pl.ds(j * CH, CH), :],
                    dst_ref=acc_ref.at[pl.ds(j * CH, CH), :],
                    send_sem=ag_send_sems.at[r],
                    recv_sem=ag_recv_sems.at[r],
                    device_id=(j,),
                    device_id_type=pl.DeviceIdType.MESH,
                ).wait_recv()

        mod = jnp.dot(temb_ref[...], wmod_ref[...],
                      preferred_element_type=jnp.float32)
        sa, sha, ga, sm, shm, gm = [mod[:, i * D:(i + 1) * D] for i in range(6)]

        x0 = x_ref[...]
        xa = _layernorm(x0) * (1.0 + sa[:, None, :]) + sha[:, None, :]
        xa2 = xa.reshape(B * S, D).astype(jnp.bfloat16)

        q = jnp.dot(xa2, wq_ref[...].astype(jnp.bfloat16),
                    preferred_element_type=jnp.float32)
        k_ = jnp.dot(xa2, wk_ref[...].astype(jnp.bfloat16),
                     preferred_element_type=jnp.float32)
        v_ = jnp.dot(xa2, wv_ref[...].astype(jnp.bfloat16),
                     preferred_element_type=jnp.float32)
        wo_bf = wo_ref[...].astype(jnp.bfloat16)

        for b in range(B):
            for h in range(H_LOC):
                qh = q[b * S:(b + 1) * S, h * DH:(h + 1) * DH].astype(jnp.bfloat16)
                kh = k_[b * S:(b + 1) * S, h * DH:(h + 1) * DH].astype(jnp.bfloat16)
                vh = v_[b * S:(b + 1) * S, h * DH:(h + 1) * DH].astype(jnp.bfloat16)
                s = jnp.dot(qh, kh.T, preferred_element_type=jnp.float32) * 0.125
                s = s - jnp.max(s, axis=-1, keepdims=True)
                p = jnp.exp(s)
                p = p / jnp.sum(p, axis=-1, keepdims=True)
                attn_ref[b * S:(b + 1) * S, h * DH:(h + 1) * DH] = jnp.dot(
                    p.astype(jnp.bfloat16), vh, preferred_element_type=jnp.float32)
            acc_ref[b * S:(b + 1) * S, :] = jnp.dot(
                attn_ref[b * S:(b + 1) * S, :].astype(jnp.bfloat16), wo_bf,
                preferred_element_type=jnp.float32).astype(jnp.bfloat16)
            if b == 0:
                pl.semaphore_wait(barrier_sem, N_DEV - 1)
            for j in range(b * (N_DEV // B), (b + 1) * (N_DEV // B)):
                start_scatter(j)

        def reduce_scat():
            total = None
            for lo in range(0, N_DEV, 8):
                for r in range(max(lo, 1), lo + 8):
                    scatter_rel(r).wait_recv()
                part = jnp.sum(scat_ref[lo:lo + 8].astype(jnp.float32), axis=0)
                total = part if total is None else total + part
            return total

        scat_ref[0] = acc_ref[pl.ds(my_i * CH, CH), :]
        reduced = reduce_scat()
        acc_ref[pl.ds(my_i * CH, CH), :] = reduced.astype(jnp.bfloat16)
        for r in range(1, N_DEV):
            gather_rdma(r).start()

        x0f = x0.reshape(B * S, D)
        n_grp = 4
        own_per = N_DEV // n_grp
        rows_per = B * S // n_grp
        wff1_bf = wff1_ref[...].astype(jnp.bfloat16)
        wff2_bf = wff2_ref[...].astype(jnp.bfloat16)
        for g in range(n_grp):
            bat = g // (n_grp // B)
            rows = slice(g * rows_per, (g + 1) * rows_per)
            for j in range(g * own_per, (g + 1) * own_per):
                wait_ag_from(j)
            x1h = x0f[rows, :] + ga[bat:bat + 1, :] * acc_ref[rows, :].astype(jnp.float32)
            x1_ref[rows, :] = x1h
            xmh = (_layernorm(x1h) * (1.0 + sm[bat:bat + 1, :])
                   + shm[bat:bat + 1, :]).astype(jnp.bfloat16)
            hh = jnp.dot(xmh, wff1_bf,
                         preferred_element_type=jnp.float32)
            hh = hh * (1.0 / (1.0 + jnp.exp(-hh)))
            p2h = jnp.dot(hh.astype(jnp.bfloat16), wff2_bf,
                          preferred_element_type=jnp.float32).astype(jnp.bfloat16)
            if g == 0:
                for r in range(1, N_DEV):
                    scatter_rel(r).wait_send()
                    gather_rdma(r).wait_send()
            acc_ref[rows, :] = p2h
            for j in range(g * own_per, (g + 1) * own_per):
                start_scatter(j)

        scat_ref[0] = acc_ref[pl.ds(my_i * CH, CH), :]
        reduced2 = reduce_scat()
        acc_ref[pl.ds(my_i * CH, CH), :] = reduced2.astype(jnp.bfloat16)
        for r in range(1, N_DEV):
            gather_rdma(r).start()

        for g in range(n_grp):
            bat = g // (n_grp // B)
            s0 = (g % (n_grp // B)) * rows_per
            rows = slice(g * rows_per, (g + 1) * rows_per)
            for j in range(g * own_per, (g + 1) * own_per):
                wait_ag_from(j)
            out_ref[bat, s0:s0 + rows_per, :] = (
                x1_ref[rows, :]
                + gm[bat:bat + 1, :] * acc_ref[rows, :].astype(jnp.float32))

        for r in range(1, N_DEV):
            scatter_rel(r).wait_send()
            gather_rdma(r).wait_send()

    vmem = pl.BlockSpec(memory_space=pltpu.VMEM)
    return pl.pallas_call(
        body,
        out_shape=jax.ShapeDtypeStruct((B, S, D), jnp.float32),
        in_specs=[vmem] * 9,
        out_specs=vmem,
        scratch_shapes=[
            pltpu.VMEM((B * S, DQ), jnp.float32),
            pltpu.VMEM((B * S, D), jnp.bfloat16),
            pltpu.VMEM((B * S, D), jnp.float32),
            pltpu.VMEM((N_DEV, CH, D), jnp.bfloat16),
            pltpu.SemaphoreType.DMA((N_DEV,)),
            pltpu.SemaphoreType.DMA((N_DEV,)),
            pltpu.SemaphoreType.DMA((N_DEV,)),
            pltpu.SemaphoreType.DMA((N_DEV,)),
        ],
        compiler_params=pltpu.CompilerParams(collective_id=0),
    )(x, Wq, Wk, Wv, Wo, t_emb, W_mod, W_ff1, W_ff2)
